# Initial kernel scaffold; baseline (speedup 1.0000x reference)
import jax
import jax.numpy as jnp
from jax.experimental import pallas as pl


def kernel(x, emb, W1, b1, W2, b2, W3, b3):
    raise NotImplementedError("write your pallas kernel here")


# speedup vs baseline: 12.8734x; 12.8734x over previous
import functools

import jax
import jax.numpy as jnp
from jax import lax
from jax.experimental import pallas as pl
from jax.experimental.pallas import tpu as pltpu
from jax.experimental.pallas import tpu_sc as plsc

B, L, V, D = 16384, 20, 1000000, 16
F_IN = L * D
N_IDX = B * L
NW = 32
PER_W = N_IDX // NW
CHUNK = 128
FIRE = 16
BLOCK = CHUNK * FIRE
N_BLOCKS = PER_W // BLOCK


def _gather_sc(x_flat, emb):
    mesh = plsc.VectorSubcoreMesh(core_axis_name="c", subcore_axis_name="s")

    @functools.partial(
        pl.kernel,
        out_type=jax.ShapeDtypeStruct((N_IDX, D), jnp.float32),
        mesh=mesh,
        scratch_types=[
            pltpu.VMEM((PER_W,), jnp.int32),
            pltpu.VMEM((BLOCK, D), jnp.float32),
            pltpu.SemaphoreType.DMA,
        ],
        compiler_params=pltpu.CompilerParams(use_tc_tiling_on_sc=False),
    )
    def k(idx_hbm, emb_hbm, out_hbm, idx_v, rows_v, sem):
        wid = lax.axis_index("s") * 2 + lax.axis_index("c")
        base = wid * PER_W
        pltpu.sync_copy(idx_hbm.at[pl.ds(base, PER_W)], idx_v)
        for blk in range(N_BLOCKS):
            blk_base = blk * BLOCK
            descs = []
            for j in range(FIRE):
                off = blk_base + j * CHUNK
                descs.append(pltpu.async_copy(
                    emb_hbm.at[idx_v.at[pl.ds(off, CHUNK)]],
                    rows_v.at[pl.ds(j * CHUNK, CHUNK)],
                    sem))
            for dsc in descs:
                dsc.wait()
            pltpu.sync_copy(rows_v, out_hbm.at[pl.ds(base + blk_base, BLOCK)])

    return k(x_flat, emb)


def _mlp_tc(h, W1, b1, W2, b2, W3, b3):
    BM = 512

    def body(h_ref, w1_ref, b1_ref, w2_ref, b2_ref, w3_ref, b3_ref, o_ref):
        a = h_ref[...]
        z1 = jnp.dot(a, w1_ref[...], preferred_element_type=jnp.float32) + b1_ref[...]
        z1 = jnp.where(z1 > 0, z1, 0.2 * z1)
        z2 = jnp.dot(z1, w2_ref[...], preferred_element_type=jnp.float32) + b2_ref[...]
        z2 = jnp.where(z2 > 0, z2, 0.2 * z2)
        o_ref[...] = jnp.dot(z2, w3_ref[...], preferred_element_type=jnp.float32) + b3_ref[...]

    return pl.pallas_call(
        body,
        grid=(B // BM,),
        in_specs=[
            pl.BlockSpec((BM, F_IN), lambda i: (i, 0)),
            pl.BlockSpec((F_IN, 512), lambda i: (0, 0)),
            pl.BlockSpec((1, 512), lambda i: (0, 0)),
            pl.BlockSpec((512, 256), lambda i: (0, 0)),
            pl.BlockSpec((1, 256), lambda i: (0, 0)),
            pl.BlockSpec((256, 1), lambda i: (0, 0)),
            pl.BlockSpec((1, 1), lambda i: (0, 0)),
        ],
        out_specs=pl.BlockSpec((BM, 1), lambda i: (i, 0)),
        out_shape=jax.ShapeDtypeStruct((B, 1), jnp.float32),
    )(h, W1, b1.reshape(1, -1), W2, b2.reshape(1, -1), W3, b3.reshape(1, -1))


def kernel(x, emb, W1, b1, W2, b2, W3, b3):
    x_flat = x.reshape(-1).astype(jnp.int32)
    rows = _gather_sc(x_flat, emb)
    h = rows.reshape(B, F_IN)
    return _mlp_tc(h, W1, b1, W2, b2, W3, b3)
